# baseline (speedup 1.0000x reference)
import functools

import jax
import jax.numpy as jnp
from jax import lax
from jax.experimental import pallas as pl
from jax.experimental.pallas import tpu as pltpu
from jax.experimental.pallas import tpu_sc as plsc

DMODEL = 64
SCALE = 8.0

NC = 2
NS = 16
NW = NC * NS

VB = 512
RING = 4


def _tc_prep(tT, V):
    H = ((V // 2 + VB - 1) // VB) * VB
    nblk = H // VB
    vblk = V // VB

    def body(a_ref, b_ref, o_ref):
        n2 = 2 * DMODEL
        row = lax.broadcasted_iota(jnp.int32, (n2, n2), 0)
        col = lax.broadcasted_iota(jnp.int32, (n2, n2), 1)
        eye8 = jnp.where(row == col, SCALE, 0.0).astype(jnp.float32)
        g = jnp.concatenate([a_ref[...], b_ref[...]], axis=0)
        o_ref[...] = lax.dot_general(
            g, eye8, (((0,), (0,)), ((), ())),
            preferred_element_type=jnp.float32)

    return pl.pallas_call(
        body,
        grid=(nblk,),
        in_specs=[
            pl.BlockSpec((DMODEL, VB), lambda i: (0, i)),
            pl.BlockSpec((DMODEL, VB),
                         lambda i, _n=nblk, _m=vblk: (0, jnp.minimum(i + _n, _m))),
        ],
        out_specs=pl.BlockSpec((VB, 2 * DMODEL), lambda i: (i, 0)),
        out_shape=jax.ShapeDtypeStruct((H, 2 * DMODEL), jnp.float32),
    )(tT, tT), H


def _make_lookup(S, T, H):
    sw = S // NW
    mesh = plsc.VectorSubcoreMesh(core_axis_name="c", subcore_axis_name="s")
    buf_t = pltpu.VMEM((sw, DMODEL), jnp.float32)

    @functools.partial(
        pl.kernel,
        mesh=mesh,
        compiler_params=pltpu.CompilerParams(use_tc_tiling_on_sc=False),
        out_type=jax.ShapeDtypeStruct((S, T, 2 * DMODEL), jnp.float32),
        scratch_types=[
            pltpu.VMEM((T, sw), jnp.int32),
            buf_t, buf_t, buf_t, buf_t,
            pltpu.SemaphoreType.DMA, pltpu.SemaphoreType.DMA,
            pltpu.SemaphoreType.DMA, pltpu.SemaphoreType.DMA,
            pltpu.SemaphoreType.DMA, pltpu.SemaphoreType.DMA,
            pltpu.SemaphoreType.DMA, pltpu.SemaphoreType.DMA,
        ],
    )
    def lookup(idx_hbm, z_hbm, out_hbm, idx_v,
               buf0, buf1, buf2, buf3,
               g0, g1, g2, g3, s0, s1, s2, s3):
        bufs = (buf0, buf1, buf2, buf3)
        gsems = (g0, g1, g2, g3)
        ssems = (s0, s1, s2, s3)
        wid = lax.axis_index("s") * NC + lax.axis_index("c")
        base = wid * sw
        pltpu.sync_copy(idx_hbm.at[:, pl.ds(base, sw)], idx_v)

        def xform(i, c):
            for j in range(sw // 16):
                sl = pl.ds(j * 16, 16)
                v = idx_v[i, sl]
                v2 = v + v
                idx_v[i, sl] = jnp.where(v >= H, v2 - (2 * H - 1), v2)
            return c

        lax.fori_loop(0, T, xform, 0)

        def fire_gather(t, b):
            pltpu.make_async_copy(
                z_hbm.at[idx_v.at[t]], bufs[b], gsems[b]).start()

        def wait_gather(t, b):
            pltpu.make_async_copy(
                z_hbm.at[idx_v.at[t]], bufs[b], gsems[b]).wait()

        def fire_scatter(t, b):
            pltpu.make_async_copy(
                bufs[b], out_hbm.at[pl.ds(base, sw), t, pl.ds(0, DMODEL)],
                ssems[b]).start()

        def wait_scatter(t, b):
            pltpu.make_async_copy(
                bufs[b], out_hbm.at[pl.ds(base, sw), t, pl.ds(0, DMODEL)],
                ssems[b]).wait()

        fire_gather(0, 0)

        def outer(s_, carry):
            for j in range(RING):
                t = s_ * RING + j
                jn = (j + 1) % RING

                @pl.when(t >= RING - 1)
                def _():
                    wait_scatter(t - (RING - 1), jn)

                @pl.when(t + 1 < T)
                def _():
                    fire_gather(t + 1, jn)

                wait_gather(t, j)
                fire_scatter(t, j)
            return carry

        lax.fori_loop(0, T // RING, outer, 0)

        for t in range(T - (RING - 1), T):
            wait_scatter(t, t % RING)

    return lookup


def kernel(input_sentence, table):
    S, T = input_sentence.shape
    V = table.shape[0]
    y, H = _tc_prep(table.T, V)
    z = y.reshape(2 * H, DMODEL)
    idxT = input_sentence.T
    out_pad = _make_lookup(S, T, H)(idxT, z)
    return out_pad[:, :, :DMODEL]


# speedup vs baseline: 1.0005x; 1.0005x over previous
import functools

import jax
import jax.numpy as jnp
from jax import lax
from jax.experimental import pallas as pl
from jax.experimental.pallas import tpu as pltpu
from jax.experimental.pallas import tpu_sc as plsc

DMODEL = 64
SCALE = 8.0

NC = 2
NS = 16
NW = NC * NS

VB = 512
RING = 4


def _tc_prep(tT, V):
    H = ((V // 2 + VB - 1) // VB) * VB
    nblk = H // VB
    vblk = V // VB

    def body(a_ref, b_ref, o_ref):
        n2 = 2 * DMODEL
        row = lax.broadcasted_iota(jnp.int32, (n2, n2), 0)
        col = lax.broadcasted_iota(jnp.int32, (n2, n2), 1)
        eye8 = jnp.where(row == col, SCALE, 0.0).astype(jnp.float32)
        g = jnp.concatenate([a_ref[...], b_ref[...]], axis=0)
        dn = (((0,), (0,)), ((), ()))
        half = VB // 2
        o_ref[:half] = lax.dot_general(
            g[:, :half], eye8, dn, preferred_element_type=jnp.float32)
        o_ref[half:] = lax.dot_general(
            g[:, half:], eye8, dn, preferred_element_type=jnp.float32)

    return pl.pallas_call(
        body,
        grid=(nblk,),
        in_specs=[
            pl.BlockSpec((DMODEL, VB), lambda i: (0, i)),
            pl.BlockSpec((DMODEL, VB),
                         lambda i, _n=nblk, _m=vblk: (0, jnp.minimum(i + _n, _m))),
        ],
        out_specs=pl.BlockSpec((VB, 2 * DMODEL), lambda i: (i, 0)),
        out_shape=jax.ShapeDtypeStruct((H, 2 * DMODEL), jnp.float32),
    )(tT, tT), H


def _make_lookup(S, T, H):
    sw = S // NW
    mesh = plsc.VectorSubcoreMesh(core_axis_name="c", subcore_axis_name="s")
    buf_t = pltpu.VMEM((sw, DMODEL), jnp.float32)

    @functools.partial(
        pl.kernel,
        mesh=mesh,
        compiler_params=pltpu.CompilerParams(use_tc_tiling_on_sc=False),
        out_type=jax.ShapeDtypeStruct((S, T, 2 * DMODEL), jnp.float32),
        scratch_types=[
            pltpu.VMEM((T, sw), jnp.int32),
            buf_t, buf_t, buf_t, buf_t,
            pltpu.SemaphoreType.DMA, pltpu.SemaphoreType.DMA,
            pltpu.SemaphoreType.DMA, pltpu.SemaphoreType.DMA,
            pltpu.SemaphoreType.DMA, pltpu.SemaphoreType.DMA,
            pltpu.SemaphoreType.DMA, pltpu.SemaphoreType.DMA,
        ],
    )
    def lookup(idx_hbm, z_hbm, out_hbm, idx_v,
               buf0, buf1, buf2, buf3,
               g0, g1, g2, g3, s0, s1, s2, s3):
        bufs = (buf0, buf1, buf2, buf3)
        gsems = (g0, g1, g2, g3)
        ssems = (s0, s1, s2, s3)
        wid = lax.axis_index("s") * NC + lax.axis_index("c")
        base = wid * sw
        pltpu.sync_copy(idx_hbm.at[:, pl.ds(base, sw)], idx_v)

        def xform(i, c):
            for j in range(sw // 16):
                sl = pl.ds(j * 16, 16)
                v = idx_v[i, sl]
                v2 = v + v
                idx_v[i, sl] = jnp.where(v >= H, v2 - (2 * H - 1), v2)
            return c

        lax.fori_loop(0, T, xform, 0)

        def fire_gather(t, b):
            pltpu.make_async_copy(
                z_hbm.at[idx_v.at[t]], bufs[b], gsems[b]).start()

        def wait_gather(t, b):
            pltpu.make_async_copy(
                z_hbm.at[idx_v.at[t]], bufs[b], gsems[b]).wait()

        def fire_scatter(t, b):
            pltpu.make_async_copy(
                bufs[b], out_hbm.at[pl.ds(base, sw), t, pl.ds(0, DMODEL)],
                ssems[b]).start()

        def wait_scatter(t, b):
            pltpu.make_async_copy(
                bufs[b], out_hbm.at[pl.ds(base, sw), t, pl.ds(0, DMODEL)],
                ssems[b]).wait()

        fire_gather(0, 0)

        def outer(s_, carry):
            for j in range(RING):
                t = s_ * RING + j
                jn = (j + 1) % RING

                @pl.when(t >= RING - 1)
                def _():
                    wait_scatter(t - (RING - 1), jn)

                @pl.when(t + 1 < T)
                def _():
                    fire_gather(t + 1, jn)

                wait_gather(t, j)
                fire_scatter(t, j)
            return carry

        lax.fori_loop(0, T // RING, outer, 0)

        for t in range(T - (RING - 1), T):
            wait_scatter(t, t % RING)

    return lookup


def kernel(input_sentence, table):
    S, T = input_sentence.shape
    V = table.shape[0]
    y, H = _tc_prep(table.T, V)
    z = y.reshape(2 * H, DMODEL)
    idxT = input_sentence.T
    out_pad = _make_lookup(S, T, H)(idxT, z)
    return out_pad[:, :, :DMODEL]


# speedup vs baseline: 1.8924x; 1.8914x over previous
import functools

import jax
import jax.numpy as jnp
from jax import lax
from jax.experimental import pallas as pl
from jax.experimental.pallas import tpu as pltpu
from jax.experimental.pallas import tpu_sc as plsc

DMODEL = 64
SCALE = 8.0

NC = 2
NS = 16
NW = NC * NS

VB = 4096
RING = 4


def _tc_prep(tT, V):
    H = ((V // 2 + VB - 1) // VB) * VB
    nblk = H // VB
    vblk = V // VB

    def body(a_ref, b_ref, o_ref):
        n2 = 2 * DMODEL
        row = lax.broadcasted_iota(jnp.int32, (n2, n2), 0)
        col = lax.broadcasted_iota(jnp.int32, (n2, n2), 1)
        eye8 = jnp.where(row == col, SCALE, 0.0).astype(jnp.float32)
        g = jnp.concatenate([a_ref[...], b_ref[...]], axis=0)
        dn = (((0,), (0,)), ((), ()))
        half = VB // 2
        o_ref[:half] = lax.dot_general(
            g[:, :half], eye8, dn, preferred_element_type=jnp.float32)
        o_ref[half:] = lax.dot_general(
            g[:, half:], eye8, dn, preferred_element_type=jnp.float32)

    return pl.pallas_call(
        body,
        grid=(nblk,),
        in_specs=[
            pl.BlockSpec((DMODEL, VB), lambda i: (0, i)),
            pl.BlockSpec((DMODEL, VB),
                         lambda i, _n=nblk, _m=vblk: (0, jnp.minimum(i + _n, _m))),
        ],
        out_specs=pl.BlockSpec((VB, 2 * DMODEL), lambda i: (i, 0)),
        out_shape=jax.ShapeDtypeStruct((H, 2 * DMODEL), jnp.float32),
    )(tT, tT), H


def _make_lookup(S, T, H):
    sw = S // NW
    mesh = plsc.VectorSubcoreMesh(core_axis_name="c", subcore_axis_name="s")
    buf_t = pltpu.VMEM((sw, DMODEL), jnp.float32)

    @functools.partial(
        pl.kernel,
        mesh=mesh,
        compiler_params=pltpu.CompilerParams(use_tc_tiling_on_sc=False),
        out_type=jax.ShapeDtypeStruct((S, T, 2 * DMODEL), jnp.float32),
        scratch_types=[
            pltpu.VMEM((T, sw), jnp.int32),
            buf_t, buf_t, buf_t, buf_t,
            pltpu.SemaphoreType.DMA, pltpu.SemaphoreType.DMA,
            pltpu.SemaphoreType.DMA, pltpu.SemaphoreType.DMA,
            pltpu.SemaphoreType.DMA, pltpu.SemaphoreType.DMA,
            pltpu.SemaphoreType.DMA, pltpu.SemaphoreType.DMA,
        ],
    )
    def lookup(idx_hbm, z_hbm, out_hbm, idx_v,
               buf0, buf1, buf2, buf3,
               g0, g1, g2, g3, s0, s1, s2, s3):
        bufs = (buf0, buf1, buf2, buf3)
        gsems = (g0, g1, g2, g3)
        ssems = (s0, s1, s2, s3)
        wid = lax.axis_index("s") * NC + lax.axis_index("c")
        base = wid * sw
        pltpu.sync_copy(idx_hbm.at[:, pl.ds(base, sw)], idx_v)

        def xform(i, c):
            for j in range(sw // 16):
                sl = pl.ds(j * 16, 16)
                v = idx_v[i, sl]
                v2 = v + v
                idx_v[i, sl] = jnp.where(v >= H, v2 - (2 * H - 1), v2)
            return c

        lax.fori_loop(0, T, xform, 0)

        def fire_gather(t, b):
            pltpu.make_async_copy(
                z_hbm.at[idx_v.at[t]], bufs[b], gsems[b]).start()

        def wait_gather(t, b):
            pltpu.make_async_copy(
                z_hbm.at[idx_v.at[t]], bufs[b], gsems[b]).wait()

        def fire_scatter(t, b):
            pltpu.make_async_copy(
                bufs[b], out_hbm.at[pl.ds(base, sw), t, pl.ds(0, DMODEL)],
                ssems[b]).start()

        def wait_scatter(t, b):
            pltpu.make_async_copy(
                bufs[b], out_hbm.at[pl.ds(base, sw), t, pl.ds(0, DMODEL)],
                ssems[b]).wait()

        fire_gather(0, 0)

        def outer(s_, carry):
            for j in range(RING):
                t = s_ * RING + j
                jn = (j + 1) % RING

                @pl.when(t >= RING - 1)
                def _():
                    wait_scatter(t - (RING - 1), jn)

                @pl.when(t + 1 < T)
                def _():
                    fire_gather(t + 1, jn)

                wait_gather(t, j)
                fire_scatter(t, j)
            return carry

        lax.fori_loop(0, T // RING, outer, 0)

        for t in range(T - (RING - 1), T):
            wait_scatter(t, t % RING)

    return lookup


def kernel(input_sentence, table):
    S, T = input_sentence.shape
    V = table.shape[0]
    y, H = _tc_prep(table.T, V)
    z = y.reshape(2 * H, DMODEL)
    idxT = input_sentence.T
    out_pad = _make_lookup(S, T, H)(idxT, z)
    return out_pad[:, :, :DMODEL]


# speedup vs baseline: 1.9117x; 1.0102x over previous
import functools

import jax
import jax.numpy as jnp
from jax import lax
from jax.experimental import pallas as pl
from jax.experimental.pallas import tpu as pltpu
from jax.experimental.pallas import tpu_sc as plsc

DMODEL = 64
SCALE = 8.0

NC = 2
NS = 16
NW = NC * NS

VB = 4096
RING = 4
AHEAD = 2


def _tc_prep(tT, V):
    H = ((V // 2 + VB - 1) // VB) * VB
    nblk = H // VB
    vblk = V // VB

    def body(a_ref, b_ref, o_ref):
        n2 = 2 * DMODEL
        row = lax.broadcasted_iota(jnp.int32, (n2, n2), 0)
        col = lax.broadcasted_iota(jnp.int32, (n2, n2), 1)
        eye8 = jnp.where(row == col, SCALE, 0.0).astype(jnp.float32)
        g = jnp.concatenate([a_ref[...], b_ref[...]], axis=0)
        dn = (((0,), (0,)), ((), ()))
        half = VB // 2
        o_ref[:half] = lax.dot_general(
            g[:, :half], eye8, dn, preferred_element_type=jnp.float32)
        o_ref[half:] = lax.dot_general(
            g[:, half:], eye8, dn, preferred_element_type=jnp.float32)

    return pl.pallas_call(
        body,
        grid=(nblk,),
        in_specs=[
            pl.BlockSpec((DMODEL, VB), lambda i: (0, i)),
            pl.BlockSpec((DMODEL, VB),
                         lambda i, _n=nblk, _m=vblk: (0, jnp.minimum(i + _n, _m))),
        ],
        out_specs=pl.BlockSpec((VB, 2 * DMODEL), lambda i: (i, 0)),
        out_shape=jax.ShapeDtypeStruct((H, 2 * DMODEL), jnp.float32),
    )(tT, tT), H


def _make_lookup(S, T, H):
    sw = S // NW
    mesh = plsc.VectorSubcoreMesh(core_axis_name="c", subcore_axis_name="s")
    buf_t = pltpu.VMEM((sw, DMODEL), jnp.float32)

    @functools.partial(
        pl.kernel,
        mesh=mesh,
        compiler_params=pltpu.CompilerParams(use_tc_tiling_on_sc=False),
        out_type=jax.ShapeDtypeStruct((S, T, 2 * DMODEL), jnp.float32),
        scratch_types=[
            pltpu.VMEM((T, sw), jnp.int32),
            buf_t, buf_t, buf_t, buf_t,
            pltpu.SemaphoreType.DMA, pltpu.SemaphoreType.DMA,
            pltpu.SemaphoreType.DMA, pltpu.SemaphoreType.DMA,
            pltpu.SemaphoreType.DMA, pltpu.SemaphoreType.DMA,
            pltpu.SemaphoreType.DMA, pltpu.SemaphoreType.DMA,
        ],
    )
    def lookup(idx_hbm, z_hbm, out_hbm, idx_v,
               buf0, buf1, buf2, buf3,
               g0, g1, g2, g3, s0, s1, s2, s3):
        bufs = (buf0, buf1, buf2, buf3)
        gsems = (g0, g1, g2, g3)
        ssems = (s0, s1, s2, s3)
        wid = lax.axis_index("s") * NC + lax.axis_index("c")
        base = wid * sw
        pltpu.sync_copy(idx_hbm.at[:, pl.ds(base, sw)], idx_v)

        def xform(i, c):
            for j in range(sw // 16):
                sl = pl.ds(j * 16, 16)
                v = idx_v[i, sl]
                v2 = v + v
                idx_v[i, sl] = jnp.where(v >= H, v2 - (2 * H - 1), v2)
            return c

        lax.fori_loop(0, T, xform, 0)

        def fire_gather(t, b):
            pltpu.make_async_copy(
                z_hbm.at[idx_v.at[t]], bufs[b], gsems[b]).start()

        def wait_gather(t, b):
            pltpu.make_async_copy(
                z_hbm.at[idx_v.at[t]], bufs[b], gsems[b]).wait()

        def fire_scatter(t, b):
            pltpu.make_async_copy(
                bufs[b], out_hbm.at[pl.ds(base, sw), t, pl.ds(0, DMODEL)],
                ssems[b]).start()

        def wait_scatter(t, b):
            pltpu.make_async_copy(
                bufs[b], out_hbm.at[pl.ds(base, sw), t, pl.ds(0, DMODEL)],
                ssems[b]).wait()

        for t0 in range(AHEAD):
            fire_gather(t0, t0)

        def outer(s_, carry):
            for j in range(RING):
                t = s_ * RING + j
                ja = (j + AHEAD) % RING

                @pl.when(t + AHEAD - RING >= 0)
                def _():
                    wait_scatter(t + AHEAD - RING, ja)

                @pl.when(t + AHEAD < T)
                def _():
                    fire_gather(t + AHEAD, ja)

                wait_gather(t, j)
                fire_scatter(t, j)
            return carry

        lax.fori_loop(0, T // RING, outer, 0)

        for t in range(T - AHEAD, T):
            wait_scatter(t, t % RING)

    return lookup


def kernel(input_sentence, table):
    S, T = input_sentence.shape
    V = table.shape[0]
    y, H = _tc_prep(table.T, V)
    z = y.reshape(2 * H, DMODEL)
    idxT = input_sentence.T
    out_pad = _make_lookup(S, T, H)(idxT, z)
    return out_pad[:, :, :DMODEL]


# speedup vs baseline: 2.0122x; 1.0526x over previous
import functools

import jax
import jax.numpy as jnp
from jax import lax
from jax.experimental import pallas as pl
from jax.experimental.pallas import tpu as pltpu
from jax.experimental.pallas import tpu_sc as plsc

DMODEL = 64
SCALE = 8.0

NC = 2
NS = 16
NW = NC * NS

VB = 8192
RING = 4
AHEAD = 2


def _tc_prep(tT, V):
    H = ((V // 2 + VB - 1) // VB) * VB
    nblk = H // VB
    vblk = V // VB

    def body(a_ref, b_ref, o_ref):
        n2 = 2 * DMODEL
        row = lax.broadcasted_iota(jnp.int32, (n2, n2), 0)
        col = lax.broadcasted_iota(jnp.int32, (n2, n2), 1)
        eye8 = jnp.where(row == col, SCALE, 0.0).astype(jnp.float32)
        g = jnp.concatenate([a_ref[...], b_ref[...]], axis=0)
        dn = (((0,), (0,)), ((), ()))
        half = VB // 2
        o_ref[:half] = lax.dot_general(
            g[:, :half], eye8, dn, preferred_element_type=jnp.float32)
        o_ref[half:] = lax.dot_general(
            g[:, half:], eye8, dn, preferred_element_type=jnp.float32)

    return pl.pallas_call(
        body,
        grid=(nblk,),
        in_specs=[
            pl.BlockSpec((DMODEL, VB), lambda i: (0, i)),
            pl.BlockSpec((DMODEL, VB),
                         lambda i, _n=nblk, _m=vblk: (0, jnp.minimum(i + _n, _m))),
        ],
        out_specs=pl.BlockSpec((VB, 2 * DMODEL), lambda i: (i, 0)),
        out_shape=jax.ShapeDtypeStruct((H, 2 * DMODEL), jnp.float32),
    )(tT, tT), H


def _make_lookup(S, T, H):
    sw = S // NW
    mesh = plsc.VectorSubcoreMesh(core_axis_name="c", subcore_axis_name="s")
    buf_t = pltpu.VMEM((sw, DMODEL), jnp.float32)

    @functools.partial(
        pl.kernel,
        mesh=mesh,
        compiler_params=pltpu.CompilerParams(use_tc_tiling_on_sc=False),
        out_type=jax.ShapeDtypeStruct((S, T, 2 * DMODEL), jnp.float32),
        scratch_types=[
            pltpu.VMEM((T, sw), jnp.int32),
            buf_t, buf_t, buf_t, buf_t,
            pltpu.SemaphoreType.DMA, pltpu.SemaphoreType.DMA,
            pltpu.SemaphoreType.DMA, pltpu.SemaphoreType.DMA,
            pltpu.SemaphoreType.DMA, pltpu.SemaphoreType.DMA,
            pltpu.SemaphoreType.DMA, pltpu.SemaphoreType.DMA,
        ],
    )
    def lookup(idx_hbm, z_hbm, out_hbm, idx_v,
               buf0, buf1, buf2, buf3,
               g0, g1, g2, g3, s0, s1, s2, s3):
        bufs = (buf0, buf1, buf2, buf3)
        gsems = (g0, g1, g2, g3)
        ssems = (s0, s1, s2, s3)
        wid = lax.axis_index("s") * NC + lax.axis_index("c")
        base = wid * sw
        pltpu.sync_copy(idx_hbm.at[:, pl.ds(base, sw)], idx_v)

        def xform(i, c):
            for j in range(sw // 16):
                sl = pl.ds(j * 16, 16)
                v = idx_v[i, sl]
                v2 = v + v
                idx_v[i, sl] = jnp.where(v >= H, v2 - (2 * H - 1), v2)
            return c

        lax.fori_loop(0, T, xform, 0)

        def fire_gather(t, b):
            pltpu.make_async_copy(
                z_hbm.at[idx_v.at[t]], bufs[b], gsems[b]).start()

        def wait_gather(t, b):
            pltpu.make_async_copy(
                z_hbm.at[idx_v.at[t]], bufs[b], gsems[b]).wait()

        def fire_scatter(t, b):
            pltpu.make_async_copy(
                bufs[b], out_hbm.at[pl.ds(base, sw), t, pl.ds(0, DMODEL)],
                ssems[b]).start()

        def wait_scatter(t, b):
            pltpu.make_async_copy(
                bufs[b], out_hbm.at[pl.ds(base, sw), t, pl.ds(0, DMODEL)],
                ssems[b]).wait()

        for t0 in range(AHEAD):
            fire_gather(t0, t0)

        def outer(s_, carry):
            for j in range(RING):
                t = s_ * RING + j
                ja = (j + AHEAD) % RING

                @pl.when(t + AHEAD - RING >= 0)
                def _():
                    wait_scatter(t + AHEAD - RING, ja)

                @pl.when(t + AHEAD < T)
                def _():
                    fire_gather(t + AHEAD, ja)

                wait_gather(t, j)
                fire_scatter(t, j)
            return carry

        lax.fori_loop(0, T // RING, outer, 0)

        for t in range(T - AHEAD, T):
            wait_scatter(t, t % RING)

    return lookup


def kernel(input_sentence, table):
    S, T = input_sentence.shape
    V = table.shape[0]
    y, H = _tc_prep(table.T, V)
    z = y.reshape(2 * H, DMODEL)
    idxT = input_sentence.T
    out_pad = _make_lookup(S, T, H)(idxT, z)
    return out_pad[:, :, :DMODEL]


# speedup vs baseline: 2.0244x; 1.0060x over previous
import functools

import jax
import jax.numpy as jnp
from jax import lax
from jax.experimental import pallas as pl
from jax.experimental.pallas import tpu as pltpu
from jax.experimental.pallas import tpu_sc as plsc

DMODEL = 64
SCALE = 8.0

NC = 2
NS = 16
NW = NC * NS

VB = 16384
RING = 4
AHEAD = 2


def _tc_prep(tT, V):
    H = ((V // 2 + VB - 1) // VB) * VB
    nblk = H // VB
    vblk = V // VB

    def body(a_ref, b_ref, o_ref):
        n2 = 2 * DMODEL
        row = lax.broadcasted_iota(jnp.int32, (n2, n2), 0)
        col = lax.broadcasted_iota(jnp.int32, (n2, n2), 1)
        eye8 = jnp.where(row == col, SCALE, 0.0).astype(jnp.float32)
        g = jnp.concatenate([a_ref[...], b_ref[...]], axis=0)
        dn = (((0,), (0,)), ((), ()))
        half = VB // 2
        o_ref[:half] = lax.dot_general(
            g[:, :half], eye8, dn, preferred_element_type=jnp.float32)
        o_ref[half:] = lax.dot_general(
            g[:, half:], eye8, dn, preferred_element_type=jnp.float32)

    return pl.pallas_call(
        body,
        grid=(nblk,),
        in_specs=[
            pl.BlockSpec((DMODEL, VB), lambda i: (0, i)),
            pl.BlockSpec((DMODEL, VB),
                         lambda i, _n=nblk, _m=vblk: (0, jnp.minimum(i + _n, _m))),
        ],
        out_specs=pl.BlockSpec((VB, 2 * DMODEL), lambda i: (i, 0)),
        out_shape=jax.ShapeDtypeStruct((H, 2 * DMODEL), jnp.float32),
    )(tT, tT), H


def _make_lookup(S, T, H):
    sw = S // NW
    mesh = plsc.VectorSubcoreMesh(core_axis_name="c", subcore_axis_name="s")
    buf_t = pltpu.VMEM((sw, DMODEL), jnp.float32)

    @functools.partial(
        pl.kernel,
        mesh=mesh,
        compiler_params=pltpu.CompilerParams(use_tc_tiling_on_sc=False),
        out_type=jax.ShapeDtypeStruct((S, T, 2 * DMODEL), jnp.float32),
        scratch_types=[
            pltpu.VMEM((T, sw), jnp.int32),
            buf_t, buf_t, buf_t, buf_t,
            pltpu.SemaphoreType.DMA, pltpu.SemaphoreType.DMA,
            pltpu.SemaphoreType.DMA, pltpu.SemaphoreType.DMA,
            pltpu.SemaphoreType.DMA, pltpu.SemaphoreType.DMA,
            pltpu.SemaphoreType.DMA, pltpu.SemaphoreType.DMA,
        ],
    )
    def lookup(idx_hbm, z_hbm, out_hbm, idx_v,
               buf0, buf1, buf2, buf3,
               g0, g1, g2, g3, s0, s1, s2, s3):
        bufs = (buf0, buf1, buf2, buf3)
        gsems = (g0, g1, g2, g3)
        ssems = (s0, s1, s2, s3)
        wid = lax.axis_index("s") * NC + lax.axis_index("c")
        base = wid * sw
        pltpu.sync_copy(idx_hbm.at[:, pl.ds(base, sw)], idx_v)

        def xform(i, c):
            for j in range(sw // 16):
                sl = pl.ds(j * 16, 16)
                v = idx_v[i, sl]
                v2 = v + v
                idx_v[i, sl] = jnp.where(v >= H, v2 - (2 * H - 1), v2)
            return c

        lax.fori_loop(0, T, xform, 0)

        def fire_gather(t, b):
            pltpu.make_async_copy(
                z_hbm.at[idx_v.at[t]], bufs[b], gsems[b]).start()

        def wait_gather(t, b):
            pltpu.make_async_copy(
                z_hbm.at[idx_v.at[t]], bufs[b], gsems[b]).wait()

        def fire_scatter(t, b):
            pltpu.make_async_copy(
                bufs[b], out_hbm.at[pl.ds(base, sw), t, pl.ds(0, DMODEL)],
                ssems[b]).start()

        def wait_scatter(t, b):
            pltpu.make_async_copy(
                bufs[b], out_hbm.at[pl.ds(base, sw), t, pl.ds(0, DMODEL)],
                ssems[b]).wait()

        for t0 in range(AHEAD):
            fire_gather(t0, t0)

        def outer(s_, carry):
            for j in range(RING):
                t = s_ * RING + j
                ja = (j + AHEAD) % RING

                @pl.when(t + AHEAD - RING >= 0)
                def _():
                    wait_scatter(t + AHEAD - RING, ja)

                @pl.when(t + AHEAD < T)
                def _():
                    fire_gather(t + AHEAD, ja)

                wait_gather(t, j)
                fire_scatter(t, j)
            return carry

        lax.fori_loop(0, T // RING, outer, 0)

        for t in range(T - AHEAD, T):
            wait_scatter(t, t % RING)

    return lookup


def kernel(input_sentence, table):
    S, T = input_sentence.shape
    V = table.shape[0]
    y, H = _tc_prep(table.T, V)
    z = y.reshape(2 * H, DMODEL)
    idxT = input_sentence.T
    out_pad = _make_lookup(S, T, H)(idxT, z)
    return out_pad[:, :, :DMODEL]
